# baseline (speedup 1.0000x reference)
import jax
import jax.numpy as jnp
from jax.experimental import pallas as pl
from jax.experimental.pallas import tpu as pltpu

_LEVELS = 3
_TILE = 4096
_KC = 128


def _level_body(first, last, r_ref, r2_ref, b1_ref, b2_ref, b3_ref, emn_ref,
                e2_ref, acc_ref, accout_ref, rout_ref, idx_ref, sums_ref):
    i = pl.program_id(0)

    @pl.when(i == 0)
    def _init():
        sums_ref[0, 0] = 0.0

    r = r_ref[...]
    t = r.shape[0]
    k = emn_ref.shape[0]
    r2row = r2_ref[...]

    gm = jnp.full((1, t), jnp.inf, jnp.float32)
    gidx = jnp.full((1, t), k, jnp.int32)
    for c in range(0, k, _KC):
        s2 = jax.lax.dot_general(emn_ref[c:c + _KC, :], r,
                                 (((1,), (1,)), ((), ())),
                                 preferred_element_type=jnp.float32)
        dist = (e2_ref[c:c + _KC, :] + r2row) + s2
        iota = c + jax.lax.broadcasted_iota(jnp.int32, (_KC, t), 0)
        mc = jnp.min(dist, axis=0, keepdims=True)
        idxc = jnp.min(jnp.where(dist == mc, iota, k), axis=0, keepdims=True)
        take = mc < gm
        gidx = jnp.where(take, idxc, gidx)
        gm = jnp.minimum(gm, mc)

    q = jnp.zeros_like(r)
    for c in range(0, k, _KC):
        iota = c + jax.lax.broadcasted_iota(jnp.int32, (_KC, t), 0)
        onehot = (iota == gidx).astype(jnp.float32)
        for split_ref in (b1_ref, b2_ref, b3_ref):
            q = q + jax.lax.dot_general(onehot, split_ref[c:c + _KC, :],
                                        (((0,), (0,)), ((), ())),
                                        preferred_element_type=jnp.float32)

    d = q - r
    sums_ref[0, 0] += jnp.sum(d * d)
    zq = r + d
    if first:
        accout_ref[...] = zq
    else:
        accout_ref[...] = acc_ref[...] + zq
    if not last:
        rout_ref[...] = r - zq
    idx_ref[0, :] = gidx[0, :]


def _level_call(r, r2, bsplits, emn, e2, acc, first, last):
    n, dim = r.shape
    k = emn.shape[0]
    grid = n // _TILE
    tok = pl.BlockSpec((_TILE, dim), lambda i: (i, 0))
    cbook = pl.BlockSpec((k, dim), lambda i: (0, 0))
    args = [r, r2, *bsplits, emn, e2]
    in_specs = [
        tok,
        pl.BlockSpec((1, _TILE), lambda i: (0, i)),
        cbook, cbook, cbook, cbook,
        pl.BlockSpec((k, 1), lambda i: (0, 0)),
    ]
    if not first:
        args.append(acc)
        in_specs.append(tok)

    out_specs = [tok]
    out_shape = [jax.ShapeDtypeStruct((n, dim), jnp.float32)]
    if not last:
        out_specs.append(tok)
        out_shape.append(jax.ShapeDtypeStruct((n, dim), jnp.float32))
    out_specs += [
        pl.BlockSpec((1, _TILE), lambda i: (0, i)),
        pl.BlockSpec(memory_space=pltpu.SMEM),
    ]
    out_shape += [
        jax.ShapeDtypeStruct((1, n), jnp.int32),
        jax.ShapeDtypeStruct((1, 1), jnp.float32),
    ]

    def body(*refs):
        nin = 7 if first else 8
        ins, outs = refs[:nin], refs[nin:]
        r_ref, r2_ref, b1, b2, b3, emn_ref, e2_ref = ins[:7]
        acc_ref = None if first else ins[7]
        if last:
            ao, ix, sm = outs
            ro = None
        else:
            ao, ro, ix, sm = outs
        _level_body(first, last, r_ref, r2_ref, b1, b2, b3, emn_ref, e2_ref,
                    acc_ref, ao, ro, ix, sm)

    res = pl.pallas_call(
        body,
        grid=(grid,),
        in_specs=in_specs,
        out_specs=out_specs,
        out_shape=out_shape,
        compiler_params=pltpu.CompilerParams(
            dimension_semantics=("arbitrary",),
        ),
    )(*args)
    if last:
        accout, idx, s = res
        rout = None
    else:
        accout, rout, idx, s = res
    return accout, rout, idx, s


def kernel(z_e, emb0, emb1, emb2):
    n, dim = z_e.shape
    embs = (emb0, emb1, emb2)
    e2 = tuple(jnp.sum(e ** 2, axis=1)[:, None] for e in embs)
    emns = tuple(e * jnp.float32(-2.0) for e in embs)

    def split3(e):
        b1 = e.astype(jnp.bfloat16).astype(jnp.float32)
        rem = e - b1
        b2 = rem.astype(jnp.bfloat16).astype(jnp.float32)
        b3 = rem - b2
        return (b1, b2, b3)

    bsplits = tuple(split3(e) for e in embs)

    r = z_e
    acc = None
    idx_rows = []
    sums = []
    for lvl in range(_LEVELS):
        r2 = jnp.sum(r ** 2, axis=1)[None, :]
        acc, r, idx, s = _level_call(r, r2, bsplits[lvl], emns[lvl], e2[lvl],
                                     acc, lvl == 0, lvl == _LEVELS - 1)
        idx_rows.append(idx)
        sums.append(s[0, 0])

    total = (sums[0] + sums[1]) + sums[2]
    vq_loss = total / jnp.float32(n * dim) / jnp.float32(_LEVELS)
    commitment_loss = jnp.float32(0.25) * vq_loss
    stacked = jnp.stack([ix.reshape(n) for ix in idx_rows], axis=1)
    return (acc, vq_loss, commitment_loss, stacked)


# speedup vs baseline: 1.0651x; 1.0651x over previous
import jax
import jax.numpy as jnp
from jax.experimental import pallas as pl
from jax.experimental.pallas import tpu as pltpu

_LEVELS = 3
_TILE = 4096
_KC = 256


def _level_body(first, last, r_ref, r2_ref, b1_ref, b2_ref, b3_ref, emn_ref,
                e2_ref, acc_ref, accout_ref, rout_ref, idx_ref, sums_ref):
    i = pl.program_id(0)

    @pl.when(i == 0)
    def _init():
        sums_ref[0, 0] = 0.0

    r = r_ref[...]
    t = r.shape[0]
    k = emn_ref.shape[0]
    r2row = r2_ref[...]

    gm = jnp.full((1, t), jnp.inf, jnp.float32)
    gidx = jnp.full((1, t), k, jnp.int32)
    for c in range(0, k, _KC):
        s2 = jax.lax.dot_general(emn_ref[c:c + _KC, :], r,
                                 (((1,), (1,)), ((), ())),
                                 preferred_element_type=jnp.float32)
        dist = (e2_ref[c:c + _KC, :] + r2row) + s2
        iota = c + jax.lax.broadcasted_iota(jnp.int32, (_KC, t), 0)
        mc = jnp.min(dist, axis=0, keepdims=True)
        idxc = jnp.min(jnp.where(dist == mc, iota, k), axis=0, keepdims=True)
        take = mc < gm
        gidx = jnp.where(take, idxc, gidx)
        gm = jnp.minimum(gm, mc)

    q = jnp.zeros_like(r)
    for c in range(0, k, _KC):
        iota = c + jax.lax.broadcasted_iota(jnp.int32, (_KC, t), 0)
        onehot = (iota == gidx).astype(jnp.float32)
        for split_ref in (b1_ref, b2_ref, b3_ref):
            q = q + jax.lax.dot_general(onehot, split_ref[c:c + _KC, :],
                                        (((0,), (0,)), ((), ())),
                                        preferred_element_type=jnp.float32)

    d = q - r
    sums_ref[0, 0] += jnp.sum(d * d)
    zq = r + d
    if first:
        accout_ref[...] = zq
    else:
        accout_ref[...] = acc_ref[...] + zq
    if not last:
        rout_ref[...] = r - zq
    idx_ref[0, :] = gidx[0, :]


def _level_call(r, r2, bsplits, emn, e2, acc, first, last):
    n, dim = r.shape
    k = emn.shape[0]
    grid = n // _TILE
    tok = pl.BlockSpec((_TILE, dim), lambda i: (i, 0))
    cbook = pl.BlockSpec((k, dim), lambda i: (0, 0))
    args = [r, r2, *bsplits, emn, e2]
    in_specs = [
        tok,
        pl.BlockSpec((1, _TILE), lambda i: (0, i)),
        cbook, cbook, cbook, cbook,
        pl.BlockSpec((k, 1), lambda i: (0, 0)),
    ]
    if not first:
        args.append(acc)
        in_specs.append(tok)

    out_specs = [tok]
    out_shape = [jax.ShapeDtypeStruct((n, dim), jnp.float32)]
    if not last:
        out_specs.append(tok)
        out_shape.append(jax.ShapeDtypeStruct((n, dim), jnp.float32))
    out_specs += [
        pl.BlockSpec((1, _TILE), lambda i: (0, i)),
        pl.BlockSpec(memory_space=pltpu.SMEM),
    ]
    out_shape += [
        jax.ShapeDtypeStruct((1, n), jnp.int32),
        jax.ShapeDtypeStruct((1, 1), jnp.float32),
    ]

    def body(*refs):
        nin = 7 if first else 8
        ins, outs = refs[:nin], refs[nin:]
        r_ref, r2_ref, b1, b2, b3, emn_ref, e2_ref = ins[:7]
        acc_ref = None if first else ins[7]
        if last:
            ao, ix, sm = outs
            ro = None
        else:
            ao, ro, ix, sm = outs
        _level_body(first, last, r_ref, r2_ref, b1, b2, b3, emn_ref, e2_ref,
                    acc_ref, ao, ro, ix, sm)

    res = pl.pallas_call(
        body,
        grid=(grid,),
        in_specs=in_specs,
        out_specs=out_specs,
        out_shape=out_shape,
        compiler_params=pltpu.CompilerParams(
            dimension_semantics=("arbitrary",),
        ),
    )(*args)
    if last:
        accout, idx, s = res
        rout = None
    else:
        accout, rout, idx, s = res
    return accout, rout, idx, s


def kernel(z_e, emb0, emb1, emb2):
    n, dim = z_e.shape
    embs = (emb0, emb1, emb2)
    e2 = tuple(jnp.sum(e ** 2, axis=1)[:, None] for e in embs)
    emns = tuple(e * jnp.float32(-2.0) for e in embs)

    def split3(e):
        b1 = e.astype(jnp.bfloat16).astype(jnp.float32)
        rem = e - b1
        b2 = rem.astype(jnp.bfloat16).astype(jnp.float32)
        b3 = rem - b2
        return (b1, b2, b3)

    bsplits = tuple(split3(e) for e in embs)

    r = z_e
    acc = None
    idx_rows = []
    sums = []
    for lvl in range(_LEVELS):
        r2 = jnp.sum(r ** 2, axis=1)[None, :]
        acc, r, idx, s = _level_call(r, r2, bsplits[lvl], emns[lvl], e2[lvl],
                                     acc, lvl == 0, lvl == _LEVELS - 1)
        idx_rows.append(idx)
        sums.append(s[0, 0])

    total = (sums[0] + sums[1]) + sums[2]
    vq_loss = total / jnp.float32(n * dim) / jnp.float32(_LEVELS)
    commitment_loss = jnp.float32(0.25) * vq_loss
    stacked = jnp.stack([ix.reshape(n) for ix in idx_rows], axis=1)
    return (acc, vq_loss, commitment_loss, stacked)
